# baseline (device time: 198041 ns/iter reference)
import jax
import jax.numpy as jnp
from jax import lax
from jax.experimental import pallas as pl
from jax.experimental.pallas import tpu as pltpu

N_DEV = 4
SQ = 512
D = 1024
HQ = 8
DH = 128
SKV = 2048
SCALE = 0.08838834764831843


def kernel(x, Wq, Wo, K_ext, V_ext):
    xs = x.reshape(SQ, D).astype(jnp.bfloat16)
    wq3 = (Wq * SCALE).reshape(D, HQ, DH).transpose(1, 0, 2).astype(
        jnp.bfloat16
    )
    wo3 = Wo.reshape(HQ, DH, D).astype(jnp.bfloat16)

    def body(
        x_ref, wq_ref, wo_ref, k_ref, v_ref, out_ref,
        xall, partial, rs_buf, kbuf, vbuf,
        ag_send, ag_recv, rs_send, rs_recv, sem_k, sem_v,
    ):
        me = lax.axis_index("i")
        left = (me - 1) % N_DEV
        right = (me + 1) % N_DEV
        h0 = me * HQ

        barrier = pltpu.get_barrier_semaphore()
        for nbr in (left, right):
            pl.semaphore_signal(
                barrier, inc=1,
                device_id=(nbr,), device_id_type=pl.DeviceIdType.MESH,
            )
        pl.semaphore_wait(barrier, 2)

        def ag_rdma(hop):
            chunk = (me - hop) % N_DEV
            return pltpu.make_async_remote_copy(
                src_ref=xall.at[pl.ds(chunk, 1)],
                dst_ref=xall.at[pl.ds(chunk, 1)],
                send_sem=ag_send.at[hop],
                recv_sem=ag_recv.at[hop],
                device_id=(right,),
                device_id_type=pl.DeviceIdType.MESH,
            )

        def rs_rdma(s_):
            if s_ == 0:
                src = partial.at[pl.ds((me - 1) % N_DEV, 1)]
            else:
                src = rs_buf.at[pl.ds(s_ - 1, 1)]
            return pltpu.make_async_remote_copy(
                src_ref=src,
                dst_ref=rs_buf.at[pl.ds(s_, 1)],
                send_sem=rs_send.at[s_],
                recv_sem=rs_recv.at[s_],
                device_id=(right,),
                device_id_type=pl.DeviceIdType.MESH,
            )

        def kv_dma(bj, h, slot):
            ck = pltpu.make_async_copy(
                k_ref.at[pl.ds(bj, 1), :, pl.ds(h0 + h, 1), :],
                kbuf.at[pl.ds(slot, 1)],
                sem_k.at[slot],
            )
            cv = pltpu.make_async_copy(
                v_ref.at[pl.ds(bj, 1), :, pl.ds(h0 + h, 1), :],
                vbuf.at[pl.ds(slot, 1), :, :, pl.ds(0, DH)],
                sem_v.at[slot],
            )
            return ck, cv

        def compute_batch(j):
            bj = (me - j) % N_DEV
            xb = xall[pl.ds(bj, 1)][0]

            def one_head(h, slot):
                k = kbuf[pl.ds(slot, 1)][0, :, 0, :].astype(jnp.bfloat16)
                va = vbuf[pl.ds(slot, 1)][0, :, 0, :].astype(jnp.bfloat16)
                wq_h = wq_ref[pl.ds(h, 1)][0]
                q = jnp.dot(xb, wq_h,
                            preferred_element_type=jnp.float32)
                q = q.astype(jnp.bfloat16)
                s = lax.dot_general(
                    q, k, (((1,), (1,)), ((), ())),
                    preferred_element_type=jnp.float32,
                )
                p = jnp.exp(s)
                o2 = jnp.dot(
                    p.astype(jnp.bfloat16), va,
                    preferred_element_type=jnp.float32,
                )
                o = o2[:, :DH] / o2[:, DH:DH + 1]
                wo_h = wo_ref[pl.ds(h, 1)][0]
                return jnp.dot(o.astype(jnp.bfloat16), wo_h,
                               preferred_element_type=jnp.float32)

            def pair_body(hp, pb):
                base = 2 * (hp % 2)

                @pl.when(hp < HQ // 2 - 1)
                def _():
                    nbase = 2 * ((hp + 1) % 2)
                    for par in range(2):
                        nk, nv = kv_dma(bj, 2 * hp + 2 + par, nbase + par)
                        nk.start()
                        nv.start()

                for par in range(2):
                    ck, cv = kv_dma(bj, 2 * hp + par, base + par)
                    ck.wait()
                    cv.wait()
                pb = pb + one_head(2 * hp, base)
                pb = pb + one_head(2 * hp + 1, base + 1)
                return pb

            for par in range(2):
                ck0, cv0 = kv_dma(bj, par, par)
                ck0.start()
                cv0.start()
            pb = lax.fori_loop(
                0, HQ // 2, pair_body, jnp.zeros((SQ, D), jnp.float32)
            )
            partial[pl.ds(bj, 1)] = pb.astype(jnp.bfloat16)[None]

        aug = (lax.broadcasted_iota(jnp.int32, (SKV, 1, DH), 2) == 0
               ).astype(jnp.float32)
        for sl in range(4):
            vbuf[sl, :, :, DH:] = aug

        xall[pl.ds(me, 1)] = x_ref[...][None]
        ag0 = ag_rdma(0)
        ag0.start()
        compute_batch(0)

        ag0.wait()
        ag1 = ag_rdma(1)
        ag1.start()
        compute_batch(1)
        rs0 = rs_rdma(0)
        rs0.start()

        ag1.wait()
        ag2 = ag_rdma(2)
        ag2.start()
        compute_batch(2)
        rs0.wait()
        rs_buf[0] = (
            rs_buf[0].astype(jnp.float32)
            + partial[pl.ds((me - 2) % N_DEV, 1)][0].astype(jnp.float32)
        ).astype(jnp.bfloat16)
        rs1 = rs_rdma(1)
        rs1.start()

        ag2.wait()
        compute_batch(3)
        rs1.wait()
        rs_buf[1] = (
            rs_buf[1].astype(jnp.float32)
            + partial[pl.ds((me - 3) % N_DEV, 1)][0].astype(jnp.float32)
        ).astype(jnp.bfloat16)
        rs2 = rs_rdma(2)
        rs2.start()
        rs2.wait()
        out_ref[0] = (
            rs_buf[2].astype(jnp.float32)
            + partial[pl.ds(me, 1)][0].astype(jnp.float32)
        )

    return pl.pallas_call(
        body,
        out_shape=jax.ShapeDtypeStruct((1, SQ, D), jnp.float32),
        in_specs=[
            pl.BlockSpec(memory_space=pltpu.VMEM),
            pl.BlockSpec(memory_space=pltpu.VMEM),
            pl.BlockSpec(memory_space=pltpu.VMEM),
            pl.BlockSpec(memory_space=pl.ANY),
            pl.BlockSpec(memory_space=pl.ANY),
        ],
        out_specs=pl.BlockSpec(memory_space=pltpu.VMEM),
        scratch_shapes=[
            pltpu.VMEM((N_DEV, SQ, D), jnp.bfloat16),
            pltpu.VMEM((N_DEV, SQ, D), jnp.bfloat16),
            pltpu.VMEM((N_DEV - 1, SQ, D), jnp.bfloat16),
            pltpu.VMEM((4, SKV, 1, DH), jnp.float32),
            pltpu.VMEM((4, SKV, 1, 2 * DH), jnp.float32),
            pltpu.SemaphoreType.DMA((N_DEV - 1,)),
            pltpu.SemaphoreType.DMA((N_DEV - 1,)),
            pltpu.SemaphoreType.DMA((N_DEV - 1,)),
            pltpu.SemaphoreType.DMA((N_DEV - 1,)),
            pltpu.SemaphoreType.DMA((4,)),
            pltpu.SemaphoreType.DMA((4,)),
        ],
        compiler_params=pltpu.CompilerParams(
            collective_id=0,
            vmem_limit_bytes=36 * 1024 * 1024,
        ),
    )(xs, wq3, wo3, K_ext, V_ext)


# device time: 163847 ns/iter; 1.2087x vs baseline; 1.2087x over previous
import jax
import jax.numpy as jnp
from jax import lax
from jax.experimental import pallas as pl
from jax.experimental.pallas import tpu as pltpu

N_DEV = 4
SQ = 512
D = 1024
HQ = 8
DH = 128
SKV = 2048
SCALE = 0.08838834764831843


def kernel(x, Wq, Wo, K_ext, V_ext):
    xs = x.reshape(SQ, D).astype(jnp.bfloat16)
    wq3 = (Wq * SCALE).reshape(D, HQ, DH).transpose(1, 0, 2).astype(
        jnp.bfloat16
    )
    wo3 = Wo.reshape(HQ, DH, D).astype(jnp.bfloat16)

    def body(
        x_ref, wq_ref, wo_ref, k_ref, v_ref, out_ref,
        xall, partial, rs_buf, kbuf, vbuf,
        ag_send, ag_recv, rs_send, rs_recv, sem_k, sem_v,
    ):
        me = lax.axis_index("i")
        left = (me - 1) % N_DEV
        right = (me + 1) % N_DEV
        h0 = me * HQ

        barrier = pltpu.get_barrier_semaphore()
        for nbr in (left, right):
            pl.semaphore_signal(
                barrier, inc=1,
                device_id=(nbr,), device_id_type=pl.DeviceIdType.MESH,
            )
        pl.semaphore_wait(barrier, 2)

        def ag_rdma(hop):
            chunk = (me - hop) % N_DEV
            return pltpu.make_async_remote_copy(
                src_ref=xall.at[pl.ds(chunk, 1)],
                dst_ref=xall.at[pl.ds(chunk, 1)],
                send_sem=ag_send.at[hop],
                recv_sem=ag_recv.at[hop],
                device_id=(right,),
                device_id_type=pl.DeviceIdType.MESH,
            )

        def rs_rdma(s_):
            if s_ == 0:
                src = partial.at[pl.ds((me - 1) % N_DEV, 1)]
            else:
                src = rs_buf.at[pl.ds(s_ - 1, 1)]
            return pltpu.make_async_remote_copy(
                src_ref=src,
                dst_ref=rs_buf.at[pl.ds(s_, 1)],
                send_sem=rs_send.at[s_],
                recv_sem=rs_recv.at[s_],
                device_id=(right,),
                device_id_type=pl.DeviceIdType.MESH,
            )

        def kv_dma(bj, h, slot):
            ck = pltpu.make_async_copy(
                k_ref.at[pl.ds(bj, 1), :, pl.ds(h0 + h, 1), :],
                kbuf.at[pl.ds(slot, 1)],
                sem_k.at[slot],
            )
            cv = pltpu.make_async_copy(
                v_ref.at[pl.ds(bj, 1), :, pl.ds(h0 + h, 1), :],
                vbuf.at[pl.ds(slot, 1)],
                sem_v.at[slot],
            )
            return ck, cv

        def compute_batch(j):
            bj = (me - j) % N_DEV
            xb = xall[pl.ds(bj, 1)][0]

            def one_head(h, slot):
                k = kbuf[pl.ds(slot, 1)][0, :, 0, :].astype(jnp.bfloat16)
                v = vbuf[pl.ds(slot, 1)][0, :, 0, :].astype(jnp.bfloat16)
                wq_h = wq_ref[pl.ds(h, 1)][0]
                q = jnp.dot(xb, wq_h,
                            preferred_element_type=jnp.float32)
                q = q.astype(jnp.bfloat16)
                s = lax.dot_general(
                    q, k, (((1,), (1,)), ((), ())),
                    preferred_element_type=jnp.float32,
                )
                p = jnp.exp(s)
                l = jnp.sum(p, axis=1, keepdims=True)
                o = jnp.dot(
                    p.astype(jnp.bfloat16), v,
                    preferred_element_type=jnp.float32,
                ) / l
                wo_h = wo_ref[pl.ds(h, 1)][0]
                return jnp.dot(o.astype(jnp.bfloat16), wo_h,
                               preferred_element_type=jnp.float32)

            def pair_body(hp, pb):
                base = 2 * (hp % 2)

                @pl.when(hp < HQ // 2 - 1)
                def _():
                    nbase = 2 * ((hp + 1) % 2)
                    for par in range(2):
                        nk, nv = kv_dma(bj, 2 * hp + 2 + par, nbase + par)
                        nk.start()
                        nv.start()

                for par in range(2):
                    ck, cv = kv_dma(bj, 2 * hp + par, base + par)
                    ck.wait()
                    cv.wait()
                pb = pb + one_head(2 * hp, base)
                pb = pb + one_head(2 * hp + 1, base + 1)
                return pb

            for par in range(2):
                ck0, cv0 = kv_dma(bj, par, par)
                ck0.start()
                cv0.start()
            pb = lax.fori_loop(
                0, HQ // 2, pair_body, jnp.zeros((SQ, D), jnp.float32)
            )
            partial[pl.ds(bj, 1)] = pb.astype(jnp.bfloat16)[None]

        xall[pl.ds(me, 1)] = x_ref[...][None]
        ag0 = ag_rdma(0)
        ag0.start()
        compute_batch(0)

        ag0.wait()
        ag1 = ag_rdma(1)
        ag1.start()
        compute_batch(1)
        rs0 = rs_rdma(0)
        rs0.start()

        ag1.wait()
        ag2 = ag_rdma(2)
        ag2.start()
        compute_batch(2)
        rs0.wait()
        rs_buf[0] = (
            rs_buf[0].astype(jnp.float32)
            + partial[pl.ds((me - 2) % N_DEV, 1)][0].astype(jnp.float32)
        ).astype(jnp.bfloat16)
        rs1 = rs_rdma(1)
        rs1.start()

        ag2.wait()
        compute_batch(3)
        rs1.wait()
        rs_buf[1] = (
            rs_buf[1].astype(jnp.float32)
            + partial[pl.ds((me - 3) % N_DEV, 1)][0].astype(jnp.float32)
        ).astype(jnp.bfloat16)
        rs2 = rs_rdma(2)
        rs2.start()
        rs2.wait()
        out_ref[0] = (
            rs_buf[2].astype(jnp.float32)
            + partial[pl.ds(me, 1)][0].astype(jnp.float32)
        )

    return pl.pallas_call(
        body,
        out_shape=jax.ShapeDtypeStruct((1, SQ, D), jnp.float32),
        in_specs=[
            pl.BlockSpec(memory_space=pltpu.VMEM),
            pl.BlockSpec(memory_space=pltpu.VMEM),
            pl.BlockSpec(memory_space=pltpu.VMEM),
            pl.BlockSpec(memory_space=pl.ANY),
            pl.BlockSpec(memory_space=pl.ANY),
        ],
        out_specs=pl.BlockSpec(memory_space=pltpu.VMEM),
        scratch_shapes=[
            pltpu.VMEM((N_DEV, SQ, D), jnp.bfloat16),
            pltpu.VMEM((N_DEV, SQ, D), jnp.bfloat16),
            pltpu.VMEM((N_DEV - 1, SQ, D), jnp.bfloat16),
            pltpu.VMEM((4, SKV, 1, DH), jnp.float32),
            pltpu.VMEM((4, SKV, 1, DH), jnp.float32),
            pltpu.SemaphoreType.DMA((N_DEV - 1,)),
            pltpu.SemaphoreType.DMA((N_DEV - 1,)),
            pltpu.SemaphoreType.DMA((N_DEV - 1,)),
            pltpu.SemaphoreType.DMA((N_DEV - 1,)),
            pltpu.SemaphoreType.DMA((4,)),
            pltpu.SemaphoreType.DMA((4,)),
        ],
        compiler_params=pltpu.CompilerParams(
            collective_id=0,
            vmem_limit_bytes=36 * 1024 * 1024,
        ),
    )(xs, wq3, wo3, K_ext, V_ext)


# device time: 157720 ns/iter; 1.2556x vs baseline; 1.0388x over previous
import jax
import jax.numpy as jnp
from jax import lax
from jax.experimental import pallas as pl
from jax.experimental.pallas import tpu as pltpu

N_DEV = 4
SQ = 512
D = 1024
HQ = 8
DH = 128
SKV = 2048
SCALE = 0.08838834764831843


def kernel(x, Wq, Wo, K_ext, V_ext):
    xs = x.reshape(SQ, D).astype(jnp.bfloat16)
    wq3 = (Wq * SCALE).reshape(D, HQ, DH).transpose(1, 0, 2).astype(
        jnp.bfloat16
    )
    wo3 = Wo.reshape(HQ, DH, D).astype(jnp.bfloat16)

    def body(
        x_ref, wq_ref, wo_ref, k_ref, v_ref, out_ref,
        xall, partial, rs_buf, kbuf, vbuf,
        ag_send, ag_recv, rs_send, rs_recv, sem_k, sem_v,
    ):
        me = lax.axis_index("i")
        left = (me - 1) % N_DEV
        right = (me + 1) % N_DEV
        h0 = me * HQ


        def ag_rdma(hop):
            chunk = (me - hop) % N_DEV
            return pltpu.make_async_remote_copy(
                src_ref=xall.at[pl.ds(chunk, 1)],
                dst_ref=xall.at[pl.ds(chunk, 1)],
                send_sem=ag_send.at[hop],
                recv_sem=ag_recv.at[hop],
                device_id=(right,),
                device_id_type=pl.DeviceIdType.MESH,
            )

        def rs_rdma(s_):
            if s_ == 0:
                src = partial.at[pl.ds((me - 1) % N_DEV, 1)]
            else:
                src = rs_buf.at[pl.ds(s_ - 1, 1)]
            return pltpu.make_async_remote_copy(
                src_ref=src,
                dst_ref=rs_buf.at[pl.ds(s_, 1)],
                send_sem=rs_send.at[s_],
                recv_sem=rs_recv.at[s_],
                device_id=(right,),
                device_id_type=pl.DeviceIdType.MESH,
            )

        def kv_dma(bj, h, slot):
            ck = pltpu.make_async_copy(
                k_ref.at[pl.ds(bj, 1), :, pl.ds(h0 + h, 1), :],
                kbuf.at[pl.ds(slot, 1)],
                sem_k.at[slot],
            )
            cv = pltpu.make_async_copy(
                v_ref.at[pl.ds(bj, 1), :, pl.ds(h0 + h, 1), :],
                vbuf.at[pl.ds(slot, 1)],
                sem_v.at[slot],
            )
            return ck, cv

        for par in range(2):
            ck0, cv0 = kv_dma(me, par, par)
            ck0.start()
            cv0.start()

        barrier = pltpu.get_barrier_semaphore()
        for nbr in (left, right):
            pl.semaphore_signal(
                barrier, inc=1,
                device_id=(nbr,), device_id_type=pl.DeviceIdType.MESH,
            )
        pl.semaphore_wait(barrier, 2)

        def compute_batch(j):
            bj = (me - j) % N_DEV
            xb = xall[pl.ds(bj, 1)][0]

            def one_head(h, slot):
                k = kbuf[pl.ds(slot, 1)][0, :, 0, :].astype(jnp.bfloat16)
                v = vbuf[pl.ds(slot, 1)][0, :, 0, :].astype(jnp.bfloat16)
                wq_h = wq_ref[pl.ds(h, 1)][0]
                q = jnp.dot(xb, wq_h,
                            preferred_element_type=jnp.float32)
                q = q.astype(jnp.bfloat16)
                s = lax.dot_general(
                    q, k, (((1,), (1,)), ((), ())),
                    preferred_element_type=jnp.float32,
                )
                p = jnp.exp(s)
                l = jnp.sum(p, axis=1, keepdims=True)
                o = jnp.dot(
                    p.astype(jnp.bfloat16), v,
                    preferred_element_type=jnp.float32,
                ) / l
                wo_h = wo_ref[pl.ds(h, 1)][0]
                return jnp.dot(o.astype(jnp.bfloat16), wo_h,
                               preferred_element_type=jnp.float32)

            def pair_body(hp, pb):
                base = 2 * (hp % 2)

                @pl.when(hp < HQ // 2 - 1)
                def _():
                    nbase = 2 * ((hp + 1) % 2)
                    for par in range(2):
                        nk, nv = kv_dma(bj, 2 * hp + 2 + par, nbase + par)
                        nk.start()
                        nv.start()

                if j < N_DEV - 1:
                    @pl.when(hp == HQ // 2 - 1)
                    def _():
                        bn = (me - (j + 1)) % N_DEV
                        for par in range(2):
                            nk, nv = kv_dma(bn, par, par)
                            nk.start()
                            nv.start()

                for par in range(2):
                    ck, cv = kv_dma(bj, 2 * hp + par, base + par)
                    ck.wait()
                    cv.wait()
                pb = pb + one_head(2 * hp, base)
                pb = pb + one_head(2 * hp + 1, base + 1)
                return pb

            pb = lax.fori_loop(
                0, HQ // 2, pair_body, jnp.zeros((SQ, D), jnp.float32)
            )
            partial[pl.ds(bj, 1)] = pb.astype(jnp.bfloat16)[None]

        xall[pl.ds(me, 1)] = x_ref[...][None]
        ag0 = ag_rdma(0)
        ag0.start()
        compute_batch(0)

        ag0.wait()
        ag1 = ag_rdma(1)
        ag1.start()
        compute_batch(1)
        rs0 = rs_rdma(0)
        rs0.start()

        ag1.wait()
        ag2 = ag_rdma(2)
        ag2.start()
        compute_batch(2)
        rs0.wait()
        rs_buf[0] = (
            rs_buf[0].astype(jnp.float32)
            + partial[pl.ds((me - 2) % N_DEV, 1)][0].astype(jnp.float32)
        ).astype(jnp.bfloat16)
        rs1 = rs_rdma(1)
        rs1.start()

        ag2.wait()
        compute_batch(3)
        rs1.wait()
        rs_buf[1] = (
            rs_buf[1].astype(jnp.float32)
            + partial[pl.ds((me - 3) % N_DEV, 1)][0].astype(jnp.float32)
        ).astype(jnp.bfloat16)
        rs2 = rs_rdma(2)
        rs2.start()
        rs2.wait()
        out_ref[0] = (
            rs_buf[2].astype(jnp.float32)
            + partial[pl.ds(me, 1)][0].astype(jnp.float32)
        )

    return pl.pallas_call(
        body,
        out_shape=jax.ShapeDtypeStruct((1, SQ, D), jnp.float32),
        in_specs=[
            pl.BlockSpec(memory_space=pltpu.VMEM),
            pl.BlockSpec(memory_space=pltpu.VMEM),
            pl.BlockSpec(memory_space=pltpu.VMEM),
            pl.BlockSpec(memory_space=pl.ANY),
            pl.BlockSpec(memory_space=pl.ANY),
        ],
        out_specs=pl.BlockSpec(memory_space=pltpu.VMEM),
        scratch_shapes=[
            pltpu.VMEM((N_DEV, SQ, D), jnp.bfloat16),
            pltpu.VMEM((N_DEV, SQ, D), jnp.bfloat16),
            pltpu.VMEM((N_DEV - 1, SQ, D), jnp.bfloat16),
            pltpu.VMEM((4, SKV, 1, DH), jnp.float32),
            pltpu.VMEM((4, SKV, 1, DH), jnp.float32),
            pltpu.SemaphoreType.DMA((N_DEV - 1,)),
            pltpu.SemaphoreType.DMA((N_DEV - 1,)),
            pltpu.SemaphoreType.DMA((N_DEV - 1,)),
            pltpu.SemaphoreType.DMA((N_DEV - 1,)),
            pltpu.SemaphoreType.DMA((4,)),
            pltpu.SemaphoreType.DMA((4,)),
        ],
        compiler_params=pltpu.CompilerParams(
            collective_id=0,
            vmem_limit_bytes=36 * 1024 * 1024,
        ),
    )(xs, wq3, wo3, K_ext, V_ext)


# device time: 148000 ns/iter; 1.3381x vs baseline; 1.0657x over previous
import jax
import jax.numpy as jnp
from jax import lax
from jax.experimental import pallas as pl
from jax.experimental.pallas import tpu as pltpu

N_DEV = 4
SQ = 512
D = 1024
HQ = 8
DH = 128
SKV = 2048
SCALE = 0.08838834764831843


def kernel(x, Wq, Wo, K_ext, V_ext):
    xs = x.reshape(SQ, D).astype(jnp.bfloat16)
    wq3 = (Wq * SCALE).reshape(D, HQ, DH).transpose(1, 0, 2).astype(
        jnp.bfloat16
    )
    wo4 = Wo.reshape(HQ // 2, 2 * DH, D).astype(jnp.bfloat16)

    def body(
        x_ref, wq_ref, wo_ref, k_ref, v_ref, out_ref,
        xall, partial, rs_buf, kbuf, vbuf, obuf,
        ag_send, ag_recv, rs_send, rs_recv, sem_k, sem_v,
    ):
        me = lax.axis_index("i")
        left = (me - 1) % N_DEV
        right = (me + 1) % N_DEV
        h0 = me * HQ


        def ag_rdma(hop):
            chunk = (me - hop) % N_DEV
            return pltpu.make_async_remote_copy(
                src_ref=xall.at[pl.ds(chunk, 1)],
                dst_ref=xall.at[pl.ds(chunk, 1)],
                send_sem=ag_send.at[hop],
                recv_sem=ag_recv.at[hop],
                device_id=(right,),
                device_id_type=pl.DeviceIdType.MESH,
            )

        def rs_rdma(s_):
            if s_ == 0:
                src = partial.at[pl.ds((me - 1) % N_DEV, 1)]
            else:
                src = rs_buf.at[pl.ds(s_ - 1, 1)]
            return pltpu.make_async_remote_copy(
                src_ref=src,
                dst_ref=rs_buf.at[pl.ds(s_, 1)],
                send_sem=rs_send.at[s_],
                recv_sem=rs_recv.at[s_],
                device_id=(right,),
                device_id_type=pl.DeviceIdType.MESH,
            )

        def kv_dma(bj, h, slot):
            ck = pltpu.make_async_copy(
                k_ref.at[pl.ds(bj, 1), :, pl.ds(h0 + h, 1), :],
                kbuf.at[pl.ds(slot, 1)],
                sem_k.at[slot],
            )
            cv = pltpu.make_async_copy(
                v_ref.at[pl.ds(bj, 1), :, pl.ds(h0 + h, 1), :],
                vbuf.at[pl.ds(slot, 1)],
                sem_v.at[slot],
            )
            return ck, cv

        for par in range(2):
            ck0, cv0 = kv_dma(me, par, par)
            ck0.start()
            cv0.start()

        barrier = pltpu.get_barrier_semaphore()
        for nbr in (left, right):
            pl.semaphore_signal(
                barrier, inc=1,
                device_id=(nbr,), device_id_type=pl.DeviceIdType.MESH,
            )
        pl.semaphore_wait(barrier, 2)

        def compute_batch(j):
            bj = (me - j) % N_DEV
            xb = xall[pl.ds(bj, 1)][0]

            def one_head(h, slot):
                k = kbuf[pl.ds(slot, 1)][0, :, 0, :].astype(jnp.bfloat16)
                v = vbuf[pl.ds(slot, 1)][0, :, 0, :].astype(jnp.bfloat16)
                wq_h = wq_ref[pl.ds(h, 1)][0]
                q = jnp.dot(xb, wq_h,
                            preferred_element_type=jnp.float32)
                q = q.astype(jnp.bfloat16)
                s = lax.dot_general(
                    q, k, (((1,), (1,)), ((), ())),
                    preferred_element_type=jnp.float32,
                )
                p = jnp.exp(s)
                l = jnp.sum(p, axis=1, keepdims=True)
                o = jnp.dot(
                    p.astype(jnp.bfloat16), v,
                    preferred_element_type=jnp.float32,
                ) / l
                return o.astype(jnp.bfloat16)

            def pair_body(hp, pb):
                base = 2 * (hp % 2)

                @pl.when(hp < HQ // 2 - 1)
                def _():
                    nbase = 2 * ((hp + 1) % 2)
                    for par in range(2):
                        nk, nv = kv_dma(bj, 2 * hp + 2 + par, nbase + par)
                        nk.start()
                        nv.start()

                if j < N_DEV - 1:
                    @pl.when(hp == HQ // 2 - 1)
                    def _():
                        bn = (me - (j + 1)) % N_DEV
                        for par in range(2):
                            nk, nv = kv_dma(bn, par, par)
                            nk.start()
                            nv.start()

                for par in range(2):
                    ck, cv = kv_dma(bj, 2 * hp + par, base + par)
                    ck.wait()
                    cv.wait()
                obuf[:, :DH] = one_head(2 * hp, base)
                obuf[:, DH:] = one_head(2 * hp + 1, base + 1)
                wo_p = wo_ref[pl.ds(hp, 1)][0]
                return pb + jnp.dot(obuf[...], wo_p,
                                    preferred_element_type=jnp.float32)

            pb = lax.fori_loop(
                0, HQ // 2, pair_body, jnp.zeros((SQ, D), jnp.float32)
            )
            partial[pl.ds(bj, 1)] = pb.astype(jnp.bfloat16)[None]

        xall[pl.ds(me, 1)] = x_ref[...][None]
        ag0 = ag_rdma(0)
        ag0.start()
        compute_batch(0)

        ag0.wait()
        ag1 = ag_rdma(1)
        ag1.start()
        compute_batch(1)
        rs0 = rs_rdma(0)
        rs0.start()

        ag1.wait()
        ag2 = ag_rdma(2)
        ag2.start()
        compute_batch(2)
        rs0.wait()
        rs_buf[0] = (
            rs_buf[0].astype(jnp.float32)
            + partial[pl.ds((me - 2) % N_DEV, 1)][0].astype(jnp.float32)
        ).astype(jnp.bfloat16)
        rs1 = rs_rdma(1)
        rs1.start()

        ag2.wait()
        compute_batch(3)
        rs1.wait()
        rs_buf[1] = (
            rs_buf[1].astype(jnp.float32)
            + partial[pl.ds((me - 3) % N_DEV, 1)][0].astype(jnp.float32)
        ).astype(jnp.bfloat16)
        rs2 = rs_rdma(2)
        rs2.start()
        rs2.wait()
        out_ref[0] = (
            rs_buf[2].astype(jnp.float32)
            + partial[pl.ds(me, 1)][0].astype(jnp.float32)
        )

    return pl.pallas_call(
        body,
        out_shape=jax.ShapeDtypeStruct((1, SQ, D), jnp.float32),
        in_specs=[
            pl.BlockSpec(memory_space=pltpu.VMEM),
            pl.BlockSpec(memory_space=pltpu.VMEM),
            pl.BlockSpec(memory_space=pltpu.VMEM),
            pl.BlockSpec(memory_space=pl.ANY),
            pl.BlockSpec(memory_space=pl.ANY),
        ],
        out_specs=pl.BlockSpec(memory_space=pltpu.VMEM),
        scratch_shapes=[
            pltpu.VMEM((N_DEV, SQ, D), jnp.bfloat16),
            pltpu.VMEM((N_DEV, SQ, D), jnp.bfloat16),
            pltpu.VMEM((N_DEV - 1, SQ, D), jnp.bfloat16),
            pltpu.VMEM((4, SKV, 1, DH), jnp.float32),
            pltpu.VMEM((4, SKV, 1, DH), jnp.float32),
            pltpu.VMEM((SQ, 2 * DH), jnp.bfloat16),
            pltpu.SemaphoreType.DMA((N_DEV - 1,)),
            pltpu.SemaphoreType.DMA((N_DEV - 1,)),
            pltpu.SemaphoreType.DMA((N_DEV - 1,)),
            pltpu.SemaphoreType.DMA((N_DEV - 1,)),
            pltpu.SemaphoreType.DMA((4,)),
            pltpu.SemaphoreType.DMA((4,)),
        ],
        compiler_params=pltpu.CompilerParams(
            collective_id=0,
            vmem_limit_bytes=36 * 1024 * 1024,
        ),
    )(xs, wq3, wo4, K_ext, V_ext)


# device time: 147414 ns/iter; 1.3434x vs baseline; 1.0040x over previous
import jax
import jax.numpy as jnp
from jax import lax
from jax.experimental import pallas as pl
from jax.experimental.pallas import tpu as pltpu

N_DEV = 4
SQ = 512
D = 1024
HQ = 8
DH = 128
SKV = 2048
SCALE = 0.08838834764831843


def kernel(x, Wq, Wo, K_ext, V_ext):
    xs = x.reshape(SQ, D).astype(jnp.bfloat16)
    wq4 = (Wq * SCALE).reshape(D, HQ // 2, 2 * DH).transpose(1, 0, 2).astype(
        jnp.bfloat16
    )
    wo4 = Wo.reshape(HQ // 2, 2 * DH, D).astype(jnp.bfloat16)

    def body(
        x_ref, wq_ref, wo_ref, k_ref, v_ref, out_ref,
        xall, partial, rs_buf, kbuf, vbuf, obuf, qbuf,
        ag_send, ag_recv, rs_send, rs_recv, sem_k, sem_v,
    ):
        me = lax.axis_index("i")
        left = (me - 1) % N_DEV
        right = (me + 1) % N_DEV
        h0 = me * HQ


        def ag_rdma(hop):
            chunk = (me - hop) % N_DEV
            return pltpu.make_async_remote_copy(
                src_ref=xall.at[pl.ds(chunk, 1)],
                dst_ref=xall.at[pl.ds(chunk, 1)],
                send_sem=ag_send.at[hop],
                recv_sem=ag_recv.at[hop],
                device_id=(right,),
                device_id_type=pl.DeviceIdType.MESH,
            )

        def rs_rdma(s_):
            if s_ == 0:
                src = partial.at[pl.ds((me - 1) % N_DEV, 1)]
            else:
                src = rs_buf.at[pl.ds(s_ - 1, 1)]
            return pltpu.make_async_remote_copy(
                src_ref=src,
                dst_ref=rs_buf.at[pl.ds(s_, 1)],
                send_sem=rs_send.at[s_],
                recv_sem=rs_recv.at[s_],
                device_id=(right,),
                device_id_type=pl.DeviceIdType.MESH,
            )

        def kv_dma(bj, h, slot):
            ck = pltpu.make_async_copy(
                k_ref.at[pl.ds(bj, 1), :, pl.ds(h0 + h, 1), :],
                kbuf.at[pl.ds(slot, 1)],
                sem_k.at[slot],
            )
            cv = pltpu.make_async_copy(
                v_ref.at[pl.ds(bj, 1), :, pl.ds(h0 + h, 1), :],
                vbuf.at[pl.ds(slot, 1)],
                sem_v.at[slot],
            )
            return ck, cv

        for par in range(2):
            ck0, cv0 = kv_dma(me, par, par)
            ck0.start()
            cv0.start()

        barrier = pltpu.get_barrier_semaphore()
        for nbr in (left, right):
            pl.semaphore_signal(
                barrier, inc=1,
                device_id=(nbr,), device_id_type=pl.DeviceIdType.MESH,
            )
        pl.semaphore_wait(barrier, 2)

        def compute_batch(j):
            bj = (me - j) % N_DEV
            xb = xall[pl.ds(bj, 1)][0]

            def one_head(q, slot):
                k = kbuf[pl.ds(slot, 1)][0, :, 0, :].astype(jnp.bfloat16)
                v = vbuf[pl.ds(slot, 1)][0, :, 0, :].astype(jnp.bfloat16)
                s = lax.dot_general(
                    q, k, (((1,), (1,)), ((), ())),
                    preferred_element_type=jnp.float32,
                )
                p = jnp.exp(s)
                l = jnp.sum(p, axis=1, keepdims=True)
                o = jnp.dot(
                    p.astype(jnp.bfloat16), v,
                    preferred_element_type=jnp.float32,
                ) / l
                return o.astype(jnp.bfloat16)

            def pair_body(hp, pb):
                base = 2 * (hp % 2)

                @pl.when(hp < HQ // 2 - 1)
                def _():
                    nbase = 2 * ((hp + 1) % 2)
                    for par in range(2):
                        nk, nv = kv_dma(bj, 2 * hp + 2 + par, nbase + par)
                        nk.start()
                        nv.start()

                if j < N_DEV - 1:
                    @pl.when(hp == HQ // 2 - 1)
                    def _():
                        bn = (me - (j + 1)) % N_DEV
                        for par in range(2):
                            nk, nv = kv_dma(bn, par, par)
                            nk.start()
                            nv.start()

                for par in range(2):
                    ck, cv = kv_dma(bj, 2 * hp + par, base + par)
                    ck.wait()
                    cv.wait()
                wq_p = wq_ref[pl.ds(hp, 1)][0]
                qp = jnp.dot(xb, wq_p,
                             preferred_element_type=jnp.float32)
                qbuf[...] = qp.astype(jnp.bfloat16)
                obuf[:, :DH] = one_head(qbuf[:, :DH], base)
                obuf[:, DH:] = one_head(qbuf[:, DH:], base + 1)
                wo_p = wo_ref[pl.ds(hp, 1)][0]
                return pb + jnp.dot(obuf[...], wo_p,
                                    preferred_element_type=jnp.float32)

            pb = lax.fori_loop(
                0, HQ // 2, pair_body, jnp.zeros((SQ, D), jnp.float32)
            )
            partial[pl.ds(bj, 1)] = pb.astype(jnp.bfloat16)[None]

        xall[pl.ds(me, 1)] = x_ref[...][None]
        ag0 = ag_rdma(0)
        ag0.start()
        compute_batch(0)

        ag0.wait()
        ag1 = ag_rdma(1)
        ag1.start()
        compute_batch(1)
        rs0 = rs_rdma(0)
        rs0.start()

        ag1.wait()
        ag2 = ag_rdma(2)
        ag2.start()
        compute_batch(2)
        rs0.wait()
        rs_buf[0] = (
            rs_buf[0].astype(jnp.float32)
            + partial[pl.ds((me - 2) % N_DEV, 1)][0].astype(jnp.float32)
        ).astype(jnp.bfloat16)
        rs1 = rs_rdma(1)
        rs1.start()

        ag2.wait()
        compute_batch(3)
        rs1.wait()
        rs_buf[1] = (
            rs_buf[1].astype(jnp.float32)
            + partial[pl.ds((me - 3) % N_DEV, 1)][0].astype(jnp.float32)
        ).astype(jnp.bfloat16)
        rs2 = rs_rdma(2)
        rs2.start()
        rs2.wait()
        out_ref[0] = (
            rs_buf[2].astype(jnp.float32)
            + partial[pl.ds(me, 1)][0].astype(jnp.float32)
        )

    return pl.pallas_call(
        body,
        out_shape=jax.ShapeDtypeStruct((1, SQ, D), jnp.float32),
        in_specs=[
            pl.BlockSpec(memory_space=pltpu.VMEM),
            pl.BlockSpec(memory_space=pltpu.VMEM),
            pl.BlockSpec(memory_space=pltpu.VMEM),
            pl.BlockSpec(memory_space=pl.ANY),
            pl.BlockSpec(memory_space=pl.ANY),
        ],
        out_specs=pl.BlockSpec(memory_space=pltpu.VMEM),
        scratch_shapes=[
            pltpu.VMEM((N_DEV, SQ, D), jnp.bfloat16),
            pltpu.VMEM((N_DEV, SQ, D), jnp.bfloat16),
            pltpu.VMEM((N_DEV - 1, SQ, D), jnp.bfloat16),
            pltpu.VMEM((4, SKV, 1, DH), jnp.float32),
            pltpu.VMEM((4, SKV, 1, DH), jnp.float32),
            pltpu.VMEM((SQ, 2 * DH), jnp.bfloat16),
            pltpu.VMEM((SQ, 2 * DH), jnp.bfloat16),
            pltpu.SemaphoreType.DMA((N_DEV - 1,)),
            pltpu.SemaphoreType.DMA((N_DEV - 1,)),
            pltpu.SemaphoreType.DMA((N_DEV - 1,)),
            pltpu.SemaphoreType.DMA((N_DEV - 1,)),
            pltpu.SemaphoreType.DMA((4,)),
            pltpu.SemaphoreType.DMA((4,)),
        ],
        compiler_params=pltpu.CompilerParams(
            collective_id=0,
            vmem_limit_bytes=36 * 1024 * 1024,
        ),
    )(xs, wq4, wo4, K_ext, V_ext)
